# baseline (speedup 1.0000x reference)
import functools

import numpy as np

import jax
import jax.numpy as jnp
from jax import lax
from jax.experimental import pallas as pl
from jax.experimental.pallas import tpu as pltpu
from jax.experimental.pallas import tpu_sc as plsc

_B, _L, _C = 2, 2048, 1024
_H, _D, _S, _P = 8, 128, 17, 16
_MAXF, _MINF = 16.0, 1.0
_SP = 32
_HALO = 144
_NW = 32
_CH = (_B * _L) // _NW
_W = _CH + 2 * _HALO
_LN = 16
_TLA = 512
_TLB = 512


def _silu(v):
    return v * jax.nn.sigmoid(v)


def _dot_t(a, w):
    return lax.dot_general(a, w, (((1,), (1,)), ((), ())),
                           preferred_element_type=jnp.float32)


def _stage_a_body(x_ref, ww_ref, bw_ref, wq_ref, bq_ref, wkb_ref,
                  rep_ref, summ_ref, awlo_ref, awhi_ref, sidx_ref, ent_ref):
    i = pl.program_id(0)
    xb = x_ref[...]
    wp = _silu(_dot_t(xb, ww_ref[...]) + bw_ref[0:1, :])
    freq = jax.nn.sigmoid(wp[:, 0:_H]) * (_MAXF - _MINF) + _MINF
    phase = jnp.tanh(wp[:, _H:2 * _H]) * _MAXF
    decay = jax.nn.sigmoid(wp[:, 2 * _H:3 * _H]) * 9.5 + 0.5
    favg = jnp.mean(freq, axis=1, keepdims=True)
    pavg = jnp.mean(phase, axis=1, keepdims=True)

    base = i * _TLA
    cint = lax.broadcasted_iota(jnp.int32, (_TLA, 1), 0) + (base % _L)
    centers = cint.astype(jnp.float32)
    lane = lax.broadcasted_iota(jnp.int32, (1, _SP), 1)
    smask = lane < _S
    svec = jnp.where(smask, lane.astype(jnp.float32) - 8.0, 0.0)
    u32 = jnp.abs(svec)

    spos = centers + svec * favg + pavg
    valid = (spos >= 0.0) & (spos < float(_L)) & smask
    idx = jnp.clip(spos.astype(jnp.int32), 0, _L - 1)
    sidx_ref[...] = jnp.where(valid, idx, cint)

    q = _silu(_dot_t(xb, wq_ref[...]) + bq_ref[0:1, :])
    qk = jnp.dot(q, wkb_ref[...],
                 preferred_element_type=jnp.float32)
    qn = qk * (_P ** -0.5)
    invd = 1.0 / jnp.clip(decay, 0.1, None)

    rep = rep_ref[...]
    summ = summ_ref[...]
    frep = jnp.dot(freq, rep, preferred_element_type=jnp.float32)
    qnrep = jnp.dot(qn, rep, preferred_element_type=jnp.float32)
    invdrep = jnp.dot(invd, rep, preferred_element_type=jnp.float32)

    hs = _H * _SP
    lane2 = lax.broadcasted_iota(jnp.int32, (1, hs), 1)
    s2 = lane2 & (_SP - 1)
    smask2 = s2 < _S
    u2 = jnp.where(smask2, jnp.abs(s2.astype(jnp.float32) - 8.0), 0.0)
    svec2 = jnp.where(smask2, s2.astype(jnp.float32) - 8.0, 0.0)

    spos2 = centers + svec2 * favg + pavg
    valid2 = (spos2 >= 0.0) & (spos2 < float(_L)) & smask2
    validf2 = valid2.astype(jnp.float32)

    rel2 = frep * u2
    masked = jnp.where(valid2, rel2 * qnrep, -1e30)

    umaxv = jnp.max(jnp.where(valid, u32, -1e30), axis=1, keepdims=True)
    uminv = jnp.min(jnp.where(valid, u32, 1e30), axis=1, keepdims=True)
    anyv = umaxv > -1e29
    m8 = jnp.where(qn >= 0.0, (freq * umaxv) * qn, (freq * uminv) * qn)
    m8 = jnp.where(anyv, m8, -1e30)
    mrep = jnp.dot(m8, rep, preferred_element_type=jnp.float32)

    shifted = masked - mrep
    ez = jnp.exp(shifted)
    z8 = jnp.dot(ez, summ, preferred_element_type=jnp.float32)
    pe = jnp.exp(shifted - rel2 * invdrep) * validf2
    t8 = jnp.dot(pe, summ, preferred_element_type=jnp.float32)
    r8 = 1.0 / (t8 + 1e-8 * z8)
    attn = pe * jnp.dot(r8, rep, preferred_element_type=jnp.float32)

    awlo_ref[...] = attn[:, 0:128]
    awhi_ref[...] = attn[:, 128:256]

    entp = (-jnp.sum(attn * jnp.log(attn + 1e-8))).reshape(1, 1)

    @pl.when(i == 0)
    def _():
        ent_ref[...] = entp

    @pl.when(i != 0)
    def _():
        ent_ref[...] = ent_ref[...] + entp


def _stage_a(x2, ww, bw2, wq, bq2, wkb, rep, summ):
    return pl.pallas_call(
        _stage_a_body,
        grid=(_B * _L // _TLA,),
        in_specs=[
            pl.BlockSpec((_TLA, _C), lambda i: (i, 0)),
            pl.BlockSpec((3 * _H, _C), lambda i: (0, 0)),
            pl.BlockSpec((8, 3 * _H), lambda i: (0, 0)),
            pl.BlockSpec((_H * _P, _C), lambda i: (0, 0)),
            pl.BlockSpec((8, _H * _P), lambda i: (0, 0)),
            pl.BlockSpec((_H * _P, _H), lambda i: (0, 0)),
            pl.BlockSpec((_H, _H * _SP), lambda i: (0, 0)),
            pl.BlockSpec((_H * _SP, _H), lambda i: (0, 0)),
        ],
        out_specs=[
            pl.BlockSpec((_TLA, 128), lambda i: (i, 0)),
            pl.BlockSpec((_TLA, 128), lambda i: (i, 0)),
            pl.BlockSpec((_TLA, _SP), lambda i: (i, 0)),
            pl.BlockSpec((1, 1), lambda i: (0, 0)),
        ],
        out_shape=[
            jax.ShapeDtypeStruct((_B * _L, 128), jnp.float32),
            jax.ShapeDtypeStruct((_B * _L, 128), jnp.float32),
            jax.ShapeDtypeStruct((_B * _L, _SP), jnp.int32),
            jax.ShapeDtypeStruct((1, 1), jnp.float32),
        ],
    )(x2, ww, bw2, wq, bq2, wkb, rep, summ)


def _sc_combine_body(x_hbm, sidx_hbm, awlo_hbm, awhi_hbm, out_hbm,
                     sidx_v, awlo_v, awhi_v, xwin_v, out_v):
    wid = lax.axis_index("s") * 2 + lax.axis_index("c")
    g0 = pl.multiple_of(wid * _CH, _CH)
    b = g0 // _L
    lo = g0 - b * _L
    s0 = jnp.clip(lo - _HALO, 0, _L - _W)
    gs0 = pl.multiple_of(b * _L + s0, 16)

    pltpu.sync_copy(sidx_hbm.at[pl.ds(g0, _CH)], sidx_v)
    pltpu.sync_copy(awlo_hbm.at[pl.ds(g0, _CH)], awlo_v)
    pltpu.sync_copy(awhi_hbm.at[pl.ds(g0, _CH)], awhi_v)

    for h in range(_H):
        aw_v = awlo_v if h < 4 else awhi_v
        wo = (h % 4) * _SP
        pltpu.sync_copy(x_hbm.at[pl.ds(gs0, _W), pl.ds(h * _D, _D)], xwin_v)

        def l_body(i, carry2, aw_v=aw_v, wo=wo):
            idx_a = sidx_v[i, pl.ds(0, _LN)] - s0
            idx_b = sidx_v[i, pl.ds(_LN, _LN)] - s0
            wgt_a = aw_v[i, pl.ds(wo, _LN)]
            wgt_b = aw_v[i, pl.ds(wo + _LN, _LN)]
            accs = [jnp.zeros((_LN,), jnp.float32) for _ in range(_D // _LN)]
            for s in range(_S):
                ridx = idx_a[s] if s < _LN else idx_b[s - _LN]
                w = wgt_a[s] if s < _LN else wgt_b[s - _LN]
                for j in range(_D // _LN):
                    row = xwin_v[ridx, pl.ds(j * _LN, _LN)]
                    accs[j] = accs[j] + w * row
            for j in range(_D // _LN):
                out_v[i, pl.ds(j * _LN, _LN)] = accs[j]
            return carry2

        lax.fori_loop(0, _CH, l_body, 0, unroll=4)
        pltpu.sync_copy(out_v, out_hbm.at[pl.ds(g0, _CH), pl.ds(h * _D, _D)])


def _sc_combine(x2, sidx, awlo, awhi):
    k = functools.partial(
        pl.kernel,
        mesh=plsc.VectorSubcoreMesh(core_axis_name="c", subcore_axis_name="s"),
        out_type=jax.ShapeDtypeStruct((_B * _L, _C), jnp.float32),
        scratch_types=[
            pltpu.VMEM((_CH, _SP), jnp.int32),
            pltpu.VMEM((_CH, 128), jnp.float32),
            pltpu.VMEM((_CH, 128), jnp.float32),
            pltpu.VMEM((_W, _D), jnp.float32),
            pltpu.VMEM((_CH, _D), jnp.float32),
        ],
    )(_sc_combine_body)
    return k(x2, sidx, awlo, awhi)


def _dot_t_bf(a, wbf):
    return lax.dot_general(a.astype(jnp.bfloat16), wbf,
                           (((1,), (1,)), ((), ())),
                           preferred_element_type=jnp.float32)


def _stage_b_body(o_ref, wse1_ref, b1_ref, wse2_ref, b2_ref, wo_ref,
                  out_ref):
    o = o_ref[...]
    h1 = _silu(_dot_t(o, wse1_ref[...]) + b1_ref[0:1, :])
    se = jax.nn.sigmoid(_dot_t(h1, wse2_ref[...]) + b2_ref[0:1, :])
    out_ref[...] = _dot_t(o * se, wo_ref[...])


def _stage_b(o2, wse1, b1, wse2, b2, wo):
    return pl.pallas_call(
        _stage_b_body,
        grid=(_B * _L // _TLB,),
        in_specs=[
            pl.BlockSpec((_TLB, _C), lambda i: (i, 0)),
            pl.BlockSpec((_C // 4, _C), lambda i: (0, 0)),
            pl.BlockSpec((8, _C // 4), lambda i: (0, 0)),
            pl.BlockSpec((_C, _C // 4), lambda i: (0, 0)),
            pl.BlockSpec((8, _C), lambda i: (0, 0)),
            pl.BlockSpec((_C, _C), lambda i: (0, 0)),
        ],
        out_specs=pl.BlockSpec((_TLB, _C), lambda i: (i, 0)),
        out_shape=jax.ShapeDtypeStruct((_B * _L, _C), jnp.float32),
    )(o2, wse1, b1, wse2, b2, wo)


def kernel(x, Ww, bw, Wq, bq, Wk, Wo, Wse1, bse1, Wse2, bse2):
    x2 = x.reshape(_B * _L, _C)
    eye = jnp.eye(_H, dtype=jnp.float32)
    wkb = (eye[:, None, :] * Wk[None, :, 0:1]).reshape(_H * _P, _H)
    bw2 = jnp.broadcast_to(bw[None, :], (8, 3 * _H))
    bq2 = jnp.broadcast_to(bq[None, :], (8, _H * _P))

    hs = _H * _SP
    hh = np.arange(hs) // _SP
    rep = jnp.asarray((hh[None, :] == np.arange(_H)[:, None])
                      .astype(np.float32))
    summ = jnp.asarray((hh[:, None] == np.arange(_H)[None, :])
                       .astype(np.float32))
    awlo, awhi, sidx, ent = _stage_a(x2, Ww, bw2, Wq, bq2, wkb, rep, summ)
    out_raw = _sc_combine(x2, sidx, awlo, awhi)

    b1 = jnp.broadcast_to(bse1[None, :], (8, _C // 4))
    b2 = jnp.broadcast_to(bse2[None, :], (8, _C))
    out = _stage_b(out_raw, Wse1, b1, Wse2, b2, Wo)

    neg_ent = ent[0, 0] * (-1.0 / (_B * _L * _H))
    return out.reshape(_B, _L, _C), neg_ent


# speedup vs baseline: 1.0258x; 1.0258x over previous
import functools

import numpy as np

import jax
import jax.numpy as jnp
from jax import lax
from jax.experimental import pallas as pl
from jax.experimental.pallas import tpu as pltpu
from jax.experimental.pallas import tpu_sc as plsc

_B, _L, _C = 2, 2048, 1024
_H, _D, _S, _P = 8, 128, 17, 16
_MAXF, _MINF = 16.0, 1.0
_SP = 32
_HALO = 144
_NW = 32
_CH = (_B * _L) // _NW
_W = _CH + 2 * _HALO
_LN = 16
_TLA = 512
_TLB = 512


def _silu(v):
    return v * jax.nn.sigmoid(v)


def _dot_t(a, w):
    return lax.dot_general(a, w, (((1,), (1,)), ((), ())),
                           preferred_element_type=jnp.float32)


def _stage_a_body(x_ref, ww_ref, bw_ref, wq_ref, bq_ref, wkb_ref,
                  rep_ref, summ_ref, awlo_ref, awhi_ref, sidx_ref, ent_ref):
    i = pl.program_id(0)
    xb = x_ref[...]
    wp = _silu(_dot_t(xb, ww_ref[...]) + bw_ref[0:1, :])
    freq = jax.nn.sigmoid(wp[:, 0:_H]) * (_MAXF - _MINF) + _MINF
    phase = jnp.tanh(wp[:, _H:2 * _H]) * _MAXF
    decay = jax.nn.sigmoid(wp[:, 2 * _H:3 * _H]) * 9.5 + 0.5
    favg = jnp.mean(freq, axis=1, keepdims=True)
    pavg = jnp.mean(phase, axis=1, keepdims=True)

    base = i * _TLA
    cint = lax.broadcasted_iota(jnp.int32, (_TLA, 1), 0) + (base % _L)
    centers = cint.astype(jnp.float32)
    lane = lax.broadcasted_iota(jnp.int32, (1, _SP), 1)
    smask = lane < _S
    svec = jnp.where(smask, lane.astype(jnp.float32) - 8.0, 0.0)
    u32 = jnp.abs(svec)

    spos = centers + svec * favg + pavg
    valid = (spos >= 0.0) & (spos < float(_L)) & smask
    idx = jnp.clip(spos.astype(jnp.int32), 0, _L - 1)
    sidx_ref[...] = jnp.where(valid, idx, cint)

    q = _silu(_dot_t(xb, wq_ref[...]) + bq_ref[0:1, :])
    qk = jnp.dot(q, wkb_ref[...],
                 preferred_element_type=jnp.float32)
    qn = qk * (_P ** -0.5)
    invd = 1.0 / jnp.clip(decay, 0.1, None)

    rep = rep_ref[...]
    summ = summ_ref[...]
    frep = jnp.dot(freq, rep, preferred_element_type=jnp.float32)
    qnrep = jnp.dot(qn, rep, preferred_element_type=jnp.float32)
    invdrep = jnp.dot(invd, rep, preferred_element_type=jnp.float32)

    hs = _H * _SP
    lane2 = lax.broadcasted_iota(jnp.int32, (1, hs), 1)
    s2 = lane2 & (_SP - 1)
    smask2 = s2 < _S
    u2 = jnp.where(smask2, jnp.abs(s2.astype(jnp.float32) - 8.0), 0.0)
    svec2 = jnp.where(smask2, s2.astype(jnp.float32) - 8.0, 0.0)

    spos2 = centers + svec2 * favg + pavg
    valid2 = (spos2 >= 0.0) & (spos2 < float(_L)) & smask2
    validf2 = valid2.astype(jnp.float32)

    rel2 = frep * u2
    masked = jnp.where(valid2, rel2 * qnrep, -1e30)

    umaxv = jnp.max(jnp.where(valid, u32, -1e30), axis=1, keepdims=True)
    uminv = jnp.min(jnp.where(valid, u32, 1e30), axis=1, keepdims=True)
    anyv = umaxv > -1e29
    m8 = jnp.where(qn >= 0.0, (freq * umaxv) * qn, (freq * uminv) * qn)
    m8 = jnp.where(anyv, m8, -1e30)
    mrep = jnp.dot(m8, rep, preferred_element_type=jnp.float32)

    shifted = masked - mrep
    ez = jnp.exp(shifted)
    z8 = jnp.dot(ez, summ, preferred_element_type=jnp.float32)
    pe = jnp.exp(shifted - rel2 * invdrep) * validf2
    t8 = jnp.dot(pe, summ, preferred_element_type=jnp.float32)
    r8 = 1.0 / (t8 + 1e-8 * z8)
    attn = pe * jnp.dot(r8, rep, preferred_element_type=jnp.float32)

    awlo_ref[...] = attn[:, 0:128]
    awhi_ref[...] = attn[:, 128:256]

    entp = (-jnp.sum(attn * jnp.log(attn + 1e-8))).reshape(1, 1)

    @pl.when(i == 0)
    def _():
        ent_ref[...] = entp

    @pl.when(i != 0)
    def _():
        ent_ref[...] = ent_ref[...] + entp


def _stage_a(x2, ww, bw2, wq, bq2, wkb, rep, summ):
    return pl.pallas_call(
        _stage_a_body,
        grid=(_B * _L // _TLA,),
        in_specs=[
            pl.BlockSpec((_TLA, _C), lambda i: (i, 0)),
            pl.BlockSpec((3 * _H, _C), lambda i: (0, 0)),
            pl.BlockSpec((8, 3 * _H), lambda i: (0, 0)),
            pl.BlockSpec((_H * _P, _C), lambda i: (0, 0)),
            pl.BlockSpec((8, _H * _P), lambda i: (0, 0)),
            pl.BlockSpec((_H * _P, _H), lambda i: (0, 0)),
            pl.BlockSpec((_H, _H * _SP), lambda i: (0, 0)),
            pl.BlockSpec((_H * _SP, _H), lambda i: (0, 0)),
        ],
        out_specs=[
            pl.BlockSpec((_TLA, 128), lambda i: (i, 0)),
            pl.BlockSpec((_TLA, 128), lambda i: (i, 0)),
            pl.BlockSpec((_TLA, _SP), lambda i: (i, 0)),
            pl.BlockSpec((1, 1), lambda i: (0, 0)),
        ],
        out_shape=[
            jax.ShapeDtypeStruct((_B * _L, 128), jnp.float32),
            jax.ShapeDtypeStruct((_B * _L, 128), jnp.float32),
            jax.ShapeDtypeStruct((_B * _L, _SP), jnp.int32),
            jax.ShapeDtypeStruct((1, 1), jnp.float32),
        ],
    )(x2, ww, bw2, wq, bq2, wkb, rep, summ)


def _sc_combine_body(x_hbm, sidx_hbm, awlo_hbm, awhi_hbm, out_hbm,
                     sidx_v, awlo_v, awhi_v, xwin_v, out_v):
    wid = lax.axis_index("s") * 2 + lax.axis_index("c")
    g0 = pl.multiple_of(wid * _CH, _CH)
    b = g0 // _L
    lo = g0 - b * _L
    s0 = jnp.clip(lo - _HALO, 0, _L - _W)
    gs0 = pl.multiple_of(b * _L + s0, 16)

    pltpu.sync_copy(sidx_hbm.at[pl.ds(g0, _CH)], sidx_v)
    pltpu.sync_copy(awlo_hbm.at[pl.ds(g0, _CH)], awlo_v)
    pltpu.sync_copy(awhi_hbm.at[pl.ds(g0, _CH)], awhi_v)

    for h in range(_H):
        aw_v = awlo_v if h < 4 else awhi_v
        wo = (h % 4) * _SP
        pltpu.sync_copy(x_hbm.at[pl.ds(gs0, _W), pl.ds(h * _D, _D)], xwin_v)

        def l_body(i, carry2, aw_v=aw_v, wo=wo):
            idx_a = sidx_v[i, pl.ds(0, _LN)] - s0
            idx_b = sidx_v[i, pl.ds(_LN, _LN)] - s0
            wgt_a = aw_v[i, pl.ds(wo, _LN)]
            wgt_b = aw_v[i, pl.ds(wo + _LN, _LN)]
            accs = [jnp.zeros((_LN,), jnp.float32) for _ in range(_D // _LN)]
            for s in range(_S):
                ridx = idx_a[s] if s < _LN else idx_b[s - _LN]
                w = wgt_a[s] if s < _LN else wgt_b[s - _LN]
                for j in range(_D // _LN):
                    row = xwin_v[ridx, pl.ds(j * _LN, _LN)]
                    accs[j] = accs[j] + w * row
            for j in range(_D // _LN):
                out_v[i, pl.ds(j * _LN, _LN)] = accs[j]
            return carry2

        lax.fori_loop(0, _CH, l_body, 0, unroll=2)
        pltpu.sync_copy(out_v, out_hbm.at[pl.ds(g0, _CH), pl.ds(h * _D, _D)])


def _sc_combine(x2, sidx, awlo, awhi):
    k = functools.partial(
        pl.kernel,
        mesh=plsc.VectorSubcoreMesh(core_axis_name="c", subcore_axis_name="s"),
        out_type=jax.ShapeDtypeStruct((_B * _L, _C), jnp.float32),
        scratch_types=[
            pltpu.VMEM((_CH, _SP), jnp.int32),
            pltpu.VMEM((_CH, 128), jnp.float32),
            pltpu.VMEM((_CH, 128), jnp.float32),
            pltpu.VMEM((_W, _D), jnp.float32),
            pltpu.VMEM((_CH, _D), jnp.float32),
        ],
    )(_sc_combine_body)
    return k(x2, sidx, awlo, awhi)


def _dot_t_bf(a, wbf):
    return lax.dot_general(a.astype(jnp.bfloat16), wbf,
                           (((1,), (1,)), ((), ())),
                           preferred_element_type=jnp.float32)


def _stage_b_body(o_ref, wse1_ref, b1_ref, wse2_ref, b2_ref, wo_ref,
                  out_ref):
    o = o_ref[...]
    h1 = _silu(_dot_t(o, wse1_ref[...]) + b1_ref[0:1, :])
    se = jax.nn.sigmoid(_dot_t(h1, wse2_ref[...]) + b2_ref[0:1, :])
    out_ref[...] = _dot_t(o * se, wo_ref[...])


def _stage_b(o2, wse1, b1, wse2, b2, wo):
    return pl.pallas_call(
        _stage_b_body,
        grid=(_B * _L // _TLB,),
        in_specs=[
            pl.BlockSpec((_TLB, _C), lambda i: (i, 0)),
            pl.BlockSpec((_C // 4, _C), lambda i: (0, 0)),
            pl.BlockSpec((8, _C // 4), lambda i: (0, 0)),
            pl.BlockSpec((_C, _C // 4), lambda i: (0, 0)),
            pl.BlockSpec((8, _C), lambda i: (0, 0)),
            pl.BlockSpec((_C, _C), lambda i: (0, 0)),
        ],
        out_specs=pl.BlockSpec((_TLB, _C), lambda i: (i, 0)),
        out_shape=jax.ShapeDtypeStruct((_B * _L, _C), jnp.float32),
    )(o2, wse1, b1, wse2, b2, wo)


def kernel(x, Ww, bw, Wq, bq, Wk, Wo, Wse1, bse1, Wse2, bse2):
    x2 = x.reshape(_B * _L, _C)
    eye = jnp.eye(_H, dtype=jnp.float32)
    wkb = (eye[:, None, :] * Wk[None, :, 0:1]).reshape(_H * _P, _H)
    bw2 = jnp.broadcast_to(bw[None, :], (8, 3 * _H))
    bq2 = jnp.broadcast_to(bq[None, :], (8, _H * _P))

    hs = _H * _SP
    hh = np.arange(hs) // _SP
    rep = jnp.asarray((hh[None, :] == np.arange(_H)[:, None])
                      .astype(np.float32))
    summ = jnp.asarray((hh[:, None] == np.arange(_H)[None, :])
                       .astype(np.float32))
    awlo, awhi, sidx, ent = _stage_a(x2, Ww, bw2, Wq, bq2, wkb, rep, summ)
    out_raw = _sc_combine(x2, sidx, awlo, awhi)

    b1 = jnp.broadcast_to(bse1[None, :], (8, _C // 4))
    b2 = jnp.broadcast_to(bse2[None, :], (8, _C))
    out = _stage_b(out_raw, Wse1, b1, Wse2, b2, Wo)

    neg_ent = ent[0, 0] * (-1.0 / (_B * _L * _H))
    return out.reshape(_B, _L, _C), neg_ent


# speedup vs baseline: 1.0266x; 1.0008x over previous
import functools

import numpy as np

import jax
import jax.numpy as jnp
from jax import lax
from jax.experimental import pallas as pl
from jax.experimental.pallas import tpu as pltpu
from jax.experimental.pallas import tpu_sc as plsc

_B, _L, _C = 2, 2048, 1024
_H, _D, _S, _P = 8, 128, 17, 16
_MAXF, _MINF = 16.0, 1.0
_SP = 32
_HALO = 144
_NW = 32
_CH = (_B * _L) // _NW
_W = _CH + 2 * _HALO
_LN = 16
_TLA = 1024
_TLB = 1024


def _silu(v):
    return v * jax.nn.sigmoid(v)


def _dot_t(a, w):
    return lax.dot_general(a, w, (((1,), (1,)), ((), ())),
                           preferred_element_type=jnp.float32)


def _stage_a_body(x_ref, ww_ref, bw_ref, wq_ref, bq_ref, wkb_ref,
                  rep_ref, summ_ref, awlo_ref, awhi_ref, sidx_ref, ent_ref):
    i = pl.program_id(0)
    xb = x_ref[...]
    wp = _silu(_dot_t(xb, ww_ref[...]) + bw_ref[0:1, :])
    freq = jax.nn.sigmoid(wp[:, 0:_H]) * (_MAXF - _MINF) + _MINF
    phase = jnp.tanh(wp[:, _H:2 * _H]) * _MAXF
    decay = jax.nn.sigmoid(wp[:, 2 * _H:3 * _H]) * 9.5 + 0.5
    favg = jnp.mean(freq, axis=1, keepdims=True)
    pavg = jnp.mean(phase, axis=1, keepdims=True)

    base = i * _TLA
    cint = lax.broadcasted_iota(jnp.int32, (_TLA, 1), 0) + (base % _L)
    centers = cint.astype(jnp.float32)
    lane = lax.broadcasted_iota(jnp.int32, (1, _SP), 1)
    smask = lane < _S
    svec = jnp.where(smask, lane.astype(jnp.float32) - 8.0, 0.0)
    u32 = jnp.abs(svec)

    spos = centers + svec * favg + pavg
    valid = (spos >= 0.0) & (spos < float(_L)) & smask
    idx = jnp.clip(spos.astype(jnp.int32), 0, _L - 1)
    sidx_ref[...] = jnp.where(valid, idx, cint)

    q = _silu(_dot_t(xb, wq_ref[...]) + bq_ref[0:1, :])
    qk = jnp.dot(q, wkb_ref[...],
                 preferred_element_type=jnp.float32)
    qn = qk * (_P ** -0.5)
    invd = 1.0 / jnp.clip(decay, 0.1, None)

    rep = rep_ref[...]
    summ = summ_ref[...]
    frep = jnp.dot(freq, rep, preferred_element_type=jnp.float32)
    qnrep = jnp.dot(qn, rep, preferred_element_type=jnp.float32)
    invdrep = jnp.dot(invd, rep, preferred_element_type=jnp.float32)

    hs = _H * _SP
    lane2 = lax.broadcasted_iota(jnp.int32, (1, hs), 1)
    s2 = lane2 & (_SP - 1)
    smask2 = s2 < _S
    u2 = jnp.where(smask2, jnp.abs(s2.astype(jnp.float32) - 8.0), 0.0)
    svec2 = jnp.where(smask2, s2.astype(jnp.float32) - 8.0, 0.0)

    spos2 = centers + svec2 * favg + pavg
    valid2 = (spos2 >= 0.0) & (spos2 < float(_L)) & smask2
    validf2 = valid2.astype(jnp.float32)

    rel2 = frep * u2
    masked = jnp.where(valid2, rel2 * qnrep, -1e30)

    umaxv = jnp.max(jnp.where(valid, u32, -1e30), axis=1, keepdims=True)
    uminv = jnp.min(jnp.where(valid, u32, 1e30), axis=1, keepdims=True)
    anyv = umaxv > -1e29
    m8 = jnp.where(qn >= 0.0, (freq * umaxv) * qn, (freq * uminv) * qn)
    m8 = jnp.where(anyv, m8, -1e30)
    mrep = jnp.dot(m8, rep, preferred_element_type=jnp.float32)

    shifted = masked - mrep
    ez = jnp.exp(shifted)
    z8 = jnp.dot(ez, summ, preferred_element_type=jnp.float32)
    pe = jnp.exp(shifted - rel2 * invdrep) * validf2
    t8 = jnp.dot(pe, summ, preferred_element_type=jnp.float32)
    r8 = 1.0 / (t8 + 1e-8 * z8)
    attn = pe * jnp.dot(r8, rep, preferred_element_type=jnp.float32)

    awlo_ref[...] = attn[:, 0:128]
    awhi_ref[...] = attn[:, 128:256]

    entp = (-jnp.sum(attn * jnp.log(attn + 1e-8))).reshape(1, 1)

    @pl.when(i == 0)
    def _():
        ent_ref[...] = entp

    @pl.when(i != 0)
    def _():
        ent_ref[...] = ent_ref[...] + entp


def _stage_a(x2, ww, bw2, wq, bq2, wkb, rep, summ):
    return pl.pallas_call(
        _stage_a_body,
        grid=(_B * _L // _TLA,),
        in_specs=[
            pl.BlockSpec((_TLA, _C), lambda i: (i, 0)),
            pl.BlockSpec((3 * _H, _C), lambda i: (0, 0)),
            pl.BlockSpec((8, 3 * _H), lambda i: (0, 0)),
            pl.BlockSpec((_H * _P, _C), lambda i: (0, 0)),
            pl.BlockSpec((8, _H * _P), lambda i: (0, 0)),
            pl.BlockSpec((_H * _P, _H), lambda i: (0, 0)),
            pl.BlockSpec((_H, _H * _SP), lambda i: (0, 0)),
            pl.BlockSpec((_H * _SP, _H), lambda i: (0, 0)),
        ],
        out_specs=[
            pl.BlockSpec((_TLA, 128), lambda i: (i, 0)),
            pl.BlockSpec((_TLA, 128), lambda i: (i, 0)),
            pl.BlockSpec((_TLA, _SP), lambda i: (i, 0)),
            pl.BlockSpec((1, 1), lambda i: (0, 0)),
        ],
        out_shape=[
            jax.ShapeDtypeStruct((_B * _L, 128), jnp.float32),
            jax.ShapeDtypeStruct((_B * _L, 128), jnp.float32),
            jax.ShapeDtypeStruct((_B * _L, _SP), jnp.int32),
            jax.ShapeDtypeStruct((1, 1), jnp.float32),
        ],
    )(x2, ww, bw2, wq, bq2, wkb, rep, summ)


def _sc_combine_body(x_hbm, sidx_hbm, awlo_hbm, awhi_hbm, out_hbm,
                     sidx_v, awlo_v, awhi_v, xwin_v, out_v):
    wid = lax.axis_index("s") * 2 + lax.axis_index("c")
    g0 = pl.multiple_of(wid * _CH, _CH)
    b = g0 // _L
    lo = g0 - b * _L
    s0 = jnp.clip(lo - _HALO, 0, _L - _W)
    gs0 = pl.multiple_of(b * _L + s0, 16)

    pltpu.sync_copy(sidx_hbm.at[pl.ds(g0, _CH)], sidx_v)
    pltpu.sync_copy(awlo_hbm.at[pl.ds(g0, _CH)], awlo_v)
    pltpu.sync_copy(awhi_hbm.at[pl.ds(g0, _CH)], awhi_v)

    for h in range(_H):
        aw_v = awlo_v if h < 4 else awhi_v
        wo = (h % 4) * _SP
        pltpu.sync_copy(x_hbm.at[pl.ds(gs0, _W), pl.ds(h * _D, _D)], xwin_v)

        def l_body(i, carry2, aw_v=aw_v, wo=wo):
            idx_a = sidx_v[i, pl.ds(0, _LN)] - s0
            idx_b = sidx_v[i, pl.ds(_LN, _LN)] - s0
            wgt_a = aw_v[i, pl.ds(wo, _LN)]
            wgt_b = aw_v[i, pl.ds(wo + _LN, _LN)]
            accs = [jnp.zeros((_LN,), jnp.float32) for _ in range(_D // _LN)]
            for s in range(_S):
                ridx = idx_a[s] if s < _LN else idx_b[s - _LN]
                w = wgt_a[s] if s < _LN else wgt_b[s - _LN]
                for j in range(_D // _LN):
                    row = xwin_v[ridx, pl.ds(j * _LN, _LN)]
                    accs[j] = accs[j] + w * row
            for j in range(_D // _LN):
                out_v[i, pl.ds(j * _LN, _LN)] = accs[j]
            return carry2

        lax.fori_loop(0, _CH, l_body, 0, unroll=2)
        pltpu.sync_copy(out_v, out_hbm.at[pl.ds(g0, _CH), pl.ds(h * _D, _D)])


def _sc_combine(x2, sidx, awlo, awhi):
    k = functools.partial(
        pl.kernel,
        mesh=plsc.VectorSubcoreMesh(core_axis_name="c", subcore_axis_name="s"),
        out_type=jax.ShapeDtypeStruct((_B * _L, _C), jnp.float32),
        scratch_types=[
            pltpu.VMEM((_CH, _SP), jnp.int32),
            pltpu.VMEM((_CH, 128), jnp.float32),
            pltpu.VMEM((_CH, 128), jnp.float32),
            pltpu.VMEM((_W, _D), jnp.float32),
            pltpu.VMEM((_CH, _D), jnp.float32),
        ],
    )(_sc_combine_body)
    return k(x2, sidx, awlo, awhi)


def _dot_t_bf(a, wbf):
    return lax.dot_general(a.astype(jnp.bfloat16), wbf,
                           (((1,), (1,)), ((), ())),
                           preferred_element_type=jnp.float32)


def _stage_b_body(o_ref, wse1_ref, b1_ref, wse2_ref, b2_ref, wo_ref,
                  out_ref):
    o = o_ref[...]
    h1 = _silu(_dot_t(o, wse1_ref[...]) + b1_ref[0:1, :])
    se = jax.nn.sigmoid(_dot_t(h1, wse2_ref[...]) + b2_ref[0:1, :])
    out_ref[...] = _dot_t(o * se, wo_ref[...])


def _stage_b(o2, wse1, b1, wse2, b2, wo):
    return pl.pallas_call(
        _stage_b_body,
        grid=(_B * _L // _TLB,),
        in_specs=[
            pl.BlockSpec((_TLB, _C), lambda i: (i, 0)),
            pl.BlockSpec((_C // 4, _C), lambda i: (0, 0)),
            pl.BlockSpec((8, _C // 4), lambda i: (0, 0)),
            pl.BlockSpec((_C, _C // 4), lambda i: (0, 0)),
            pl.BlockSpec((8, _C), lambda i: (0, 0)),
            pl.BlockSpec((_C, _C), lambda i: (0, 0)),
        ],
        out_specs=pl.BlockSpec((_TLB, _C), lambda i: (i, 0)),
        out_shape=jax.ShapeDtypeStruct((_B * _L, _C), jnp.float32),
    )(o2, wse1, b1, wse2, b2, wo)


def kernel(x, Ww, bw, Wq, bq, Wk, Wo, Wse1, bse1, Wse2, bse2):
    x2 = x.reshape(_B * _L, _C)
    eye = jnp.eye(_H, dtype=jnp.float32)
    wkb = (eye[:, None, :] * Wk[None, :, 0:1]).reshape(_H * _P, _H)
    bw2 = jnp.broadcast_to(bw[None, :], (8, 3 * _H))
    bq2 = jnp.broadcast_to(bq[None, :], (8, _H * _P))

    hs = _H * _SP
    hh = np.arange(hs) // _SP
    rep = jnp.asarray((hh[None, :] == np.arange(_H)[:, None])
                      .astype(np.float32))
    summ = jnp.asarray((hh[:, None] == np.arange(_H)[None, :])
                       .astype(np.float32))
    awlo, awhi, sidx, ent = _stage_a(x2, Ww, bw2, Wq, bq2, wkb, rep, summ)
    out_raw = _sc_combine(x2, sidx, awlo, awhi)

    b1 = jnp.broadcast_to(bse1[None, :], (8, _C // 4))
    b2 = jnp.broadcast_to(bse2[None, :], (8, _C))
    out = _stage_b(out_raw, Wse1, b1, Wse2, b2, Wo)

    neg_ent = ent[0, 0] * (-1.0 / (_B * _L * _H))
    return out.reshape(_B, _L, _C), neg_ent
